# baseline (speedup 1.0000x reference)
import functools

import jax
import jax.numpy as jnp
from jax import lax
from jax.experimental import pallas as pl
from jax.experimental.pallas import tpu as pltpu
from jax.experimental.pallas import tpu_sc as plsc

BATCH = 16384
D_INST = 64
D_LIGHT = 32
D_APP = 32

_NC = 2
_NS = 16
NW = _NC * _NS
BPW = BATCH // NW
CHUNK = 128
NCH = BPW // CHUNK

_MESH = plsc.VectorSubcoreMesh(core_axis_name="c", subcore_axis_name="s")


@functools.partial(
    pl.kernel,
    mesh=_MESH,
    compiler_params=pltpu.CompilerParams(use_tc_tiling_on_sc=False),
    out_type=(
        jax.ShapeDtypeStruct((BATCH, D_INST), jnp.float32),
        jax.ShapeDtypeStruct((BATCH, D_LIGHT), jnp.float32),
        jax.ShapeDtypeStruct((BATCH, D_APP), jnp.float32),
    ),
    scratch_types=[
        pltpu.VMEM((BPW,), jnp.int32),
        pltpu.VMEM((BPW,), jnp.int32),
        pltpu.VMEM((BPW,), jnp.int32),
        pltpu.VMEM((BPW, D_INST), jnp.float32),
        pltpu.VMEM((BPW, D_LIGHT), jnp.float32),
        pltpu.VMEM((BPW, D_APP), jnp.float32),
        pltpu.SemaphoreType.DMA,
    ],
)
def _gather3(inst_hbm, light_hbm, frame_hbm, wi_hbm, wl_hbm, wa_hbm,
             out_i, out_l, out_a,
             idx_i, idx_l, idx_a, rows_i, rows_l, rows_a, sem):
    wid = lax.axis_index("s") * _NC + lax.axis_index("c")
    base = wid * BPW
    pltpu.sync_copy(inst_hbm.at[pl.ds(base, BPW)], idx_i)
    pltpu.sync_copy(light_hbm.at[pl.ds(base, BPW)], idx_l)
    pltpu.sync_copy(frame_hbm.at[pl.ds(base, BPW)], idx_a)
    copies = []
    for c in range(NCH):
        sl = pl.ds(c * CHUNK, CHUNK)
        copies.append(pltpu.async_copy(wi_hbm.at[idx_i.at[sl]], rows_i.at[sl], sem))
        copies.append(pltpu.async_copy(wl_hbm.at[idx_l.at[sl]], rows_l.at[sl], sem))
        copies.append(pltpu.async_copy(wa_hbm.at[idx_a.at[sl]], rows_a.at[sl], sem))
    for cp in copies:
        cp.wait()
    pltpu.sync_copy(rows_i, out_i.at[pl.ds(base, BPW)])
    pltpu.sync_copy(rows_l, out_l.at[pl.ds(base, BPW)])
    pltpu.sync_copy(rows_a, out_a.at[pl.ds(base, BPW)])


def kernel(instance_ids, light_env_ids, frame_ids, W_inst, W_light, W_app):
    inst = jnp.squeeze(instance_ids).astype(jnp.int32)
    light = jnp.squeeze(light_env_ids).astype(jnp.int32)
    frame = jnp.squeeze(frame_ids).astype(jnp.int32)
    return _gather3(inst, light, frame, W_inst, W_light, W_app)
